# baseline (speedup 1.0000x reference)
import functools

import jax
import jax.numpy as jnp
from jax import lax
from jax.experimental import pallas as pl
from jax.experimental.pallas import tpu as pltpu
from jax.experimental.pallas import tpu_sc as plsc

N = 10000
E = 160000
D = 128
G = 32
H1, C1 = 4, 256
H2, C2 = 4, 64
F1 = H1 * C1
F2 = H2 * C2
B = 64
N_PAD1 = 10240
N_PAD2 = 10240


def _mm_nodes1_body(x_ref, wl_ref, wr_ref, ol_ref, or_ref):
    x = x_ref[...]
    ol_ref[0] = jnp.dot(x, wl_ref[...], preferred_element_type=jnp.float32)
    or_ref[0] = jnp.dot(x, wr_ref[...], preferred_element_type=jnp.float32)


def _mm_nodes1(x, Wl, Wr):
    bn = 2000
    return pl.pallas_call(
        _mm_nodes1_body,
        grid=(H1, N // bn),
        in_specs=[
            pl.BlockSpec((bn, D), lambda h, i: (i, 0)),
            pl.BlockSpec((D, C1), lambda h, i: (0, h)),
            pl.BlockSpec((D, C1), lambda h, i: (0, h)),
        ],
        out_specs=[
            pl.BlockSpec((1, bn, C1), lambda h, i: (h, i, 0)),
            pl.BlockSpec((1, bn, C1), lambda h, i: (h, i, 0)),
        ],
        out_shape=[
            jax.ShapeDtypeStruct((H1, N, C1), jnp.float32),
            jax.ShapeDtypeStruct((H1, N, C1), jnp.float32),
        ],
    )(x, Wl, Wr)


def _mm_nodes2_body(h_ref, wl_ref, wr_ref, ol_ref, or_ref):
    accl = jnp.zeros((h_ref.shape[1], F2), jnp.float32)
    accr = jnp.zeros((h_ref.shape[1], F2), jnp.float32)
    for hh in range(H1):
        blk = h_ref[hh]
        accl += jnp.dot(blk, wl_ref[hh], preferred_element_type=jnp.float32)
        accr += jnp.dot(blk, wr_ref[hh], preferred_element_type=jnp.float32)
    ol_ref[...] = accl
    or_ref[...] = accr


def _mm_nodes2(h1, Wl, Wr):
    bn = 1000
    wl = Wl.reshape(H1, C1, F2)
    wr = Wr.reshape(H1, C1, F2)
    return pl.pallas_call(
        _mm_nodes2_body,
        grid=(N // bn,),
        in_specs=[
            pl.BlockSpec((H1, bn, C1), lambda i: (0, i, 0)),
            pl.BlockSpec((H1, C1, F2), lambda i: (0, 0, 0)),
            pl.BlockSpec((H1, C1, F2), lambda i: (0, 0, 0)),
        ],
        out_specs=[
            pl.BlockSpec((bn, F2), lambda i: (i, 0)),
            pl.BlockSpec((bn, F2), lambda i: (i, 0)),
        ],
        out_shape=[
            jax.ShapeDtypeStruct((N, F2), jnp.float32),
            jax.ShapeDtypeStruct((N, F2), jnp.float32),
        ],
    )(h1, wl, wr)


def _mm_edges_body(ea_ref, w_ref, o_ref):
    o_ref[0] = jnp.dot(ea_ref[...], w_ref[...],
                       preferred_element_type=jnp.float32)


def _mm_edges(ea, We, ht, c):
    be = 2000
    de = ea.shape[1]
    return pl.pallas_call(
        _mm_edges_body,
        grid=(ht, E // be),
        in_specs=[
            pl.BlockSpec((be, de), lambda h, i: (i, 0)),
            pl.BlockSpec((de, c), lambda h, i: (0, h)),
        ],
        out_specs=pl.BlockSpec((1, be, c), lambda h, i: (h, i, 0)),
        out_shape=jax.ShapeDtypeStruct((ht, E, c), jnp.float32),
    )(ea, We)


def _pool_fc_body(h_ref, bat_ref, w1_ref, b1_ref, w2_ref, b2_ref, o_ref):
    hv = h_ref[...]
    bat = bat_ref[...]
    gids = lax.broadcasted_iota(jnp.int32, (N_PAD2, G), 1)
    S = (bat == gids).astype(jnp.float32)
    xadd = lax.dot_general(S, hv, (((0,), (0,)), ((), ())),
                           preferred_element_type=jnp.float32)
    cnt = jnp.sum(S, axis=0)[:, None]
    xmean = xadd / jnp.maximum(cnt, 1.0)
    p = jnp.concatenate([xadd, xmean], axis=1)
    z = jnp.maximum(
        jnp.dot(p, w1_ref[...], preferred_element_type=jnp.float32)
        + b1_ref[...], 0.0)
    o_ref[...] = (jnp.dot(z, w2_ref[...], preferred_element_type=jnp.float32)
                  + b2_ref[...])


def _pool_fc(h2_pad, batch_pad, W_fc1, b_fc1, W_fc2, b_fc2):
    return pl.pallas_call(
        _pool_fc_body,
        in_specs=[
            pl.BlockSpec((N_PAD2, F2), lambda: (0, 0)),
            pl.BlockSpec((N_PAD2, 1), lambda: (0, 0)),
            pl.BlockSpec(W_fc1.shape, lambda: (0, 0)),
            pl.BlockSpec((1, 64), lambda: (0, 0)),
            pl.BlockSpec(W_fc2.shape, lambda: (0, 0)),
            pl.BlockSpec((1, 1), lambda: (0, 0)),
        ],
        out_specs=pl.BlockSpec((G, 1), lambda: (0, 0)),
        out_shape=jax.ShapeDtypeStruct((G, 1), jnp.float32),
    )(h2_pad, batch_pad, W_fc1, b_fc1.reshape(1, 64), W_fc2,
      b_fc2.reshape(1, 1))


def _gat_sc(xl, xr, e, lin, srcs, dsts, bounds, att, b, *,
            n_ranges, rng_sz, oblk, n_out_rows):
    HT, NN, CR = xl.shape
    att = att.reshape(HT, 1, CR)
    b = b.reshape(HT, 1, CR)
    K = CR // 16
    nh = 4 // HT
    cps = K // nh
    n_bpad = bounds.shape[0]
    mesh = plsc.VectorSubcoreMesh(core_axis_name="c", subcore_axis_name="s")

    @functools.partial(
        pl.kernel,
        out_type=jax.ShapeDtypeStruct((HT, n_out_rows, CR), jnp.float32),
        mesh=mesh,
        compiler_params=pltpu.CompilerParams(needs_layout_passes=False),
        scratch_types=[
            pltpu.VMEM((2, B), jnp.int32),
            pltpu.VMEM((2, B + 16), jnp.int32),
            pltpu.VMEM((2, 4 // HT, B + 16), jnp.float32),
            pltpu.VMEM((2, B, CR), jnp.float32),
            pltpu.VMEM((2, B, CR), jnp.float32),
            pltpu.VMEM((oblk, CR), jnp.float32),
            pltpu.VMEM((oblk, CR), jnp.float32),
            pltpu.VMEM((1, CR), jnp.float32),
            pltpu.VMEM((1, CR), jnp.float32),
            pltpu.VMEM((n_bpad,), jnp.int32),
            pltpu.SemaphoreType.DMA((2,)),
            pltpu.SemaphoreType.DMA((2,)),
        ],
    )
    def k(xl_hbm, xr_hbm, e_hbm, lin_hbm, srcs_hbm, dsts_hbm, bounds_hbm,
          att_hbm, b_hbm, out_hbm, src_v, dst_v, lin_v, e_v, xlg_v, xr_v,
          out_v, att_v, b_v, bounds_v, sem1, sem2):
        wid = lax.axis_index("s") * 2 + lax.axis_index("c")
        h = wid // n_ranges
        r = wid % n_ranges
        d0 = r * rng_sz
        d1 = d0 + rng_sz

        pltpu.sync_copy(bounds_hbm, bounds_v)
        pltpu.sync_copy(att_hbm.at[h], att_v)
        pltpu.sync_copy(b_hbm.at[h], b_v)
        p0 = bounds_v[pl.ds(r, 16)][0]
        p1 = bounds_v[pl.ds(r + 1, 16)][0]
        blk0 = p0 // B
        nblk = (p1 + B - 1) // B - blk0

        att_regs = [att_v[0, pl.ds(16 * kk, 16)] for kk in range(K)]
        zero_v = jnp.zeros((16,), jnp.float32)

        def init_outbuf():
            def ib(i, c):
                for kk in range(K):
                    sl = pl.ds(16 * kk, 16)
                    out_v[i, sl] = jnp.maximum(b_v[0, sl], 0.0)
                return c
            lax.fori_loop(0, oblk, ib, 0)

        def load_xr(wl):
            start = pl.multiple_of(jnp.minimum(wl, NN - oblk), 8)
            pltpu.sync_copy(xr_hbm.at[h, pl.ds(start, oblk)], xr_v)

        def flush(wl):
            wl = pl.multiple_of(wl, 8)
            pltpu.sync_copy(out_v, out_hbm.at[h, pl.ds(wl, oblk)])

        def wbody(wl):
            flush(wl)
            init_outbuf()
            load_xr(wl + oblk)
            return wl + oblk

        def finalize(d_cur, w_lo, dsums, accs):
            row = d_cur - w_lo
            invs = [1.0 / (dsums[s] + 1e-16) for s in range(nh)]
            for kk in range(K):
                sl = pl.ds(16 * kk, 16)
                val = accs[kk] * invs[kk // cps] + b_v[0, sl]
                out_v[row, sl] = jnp.maximum(val, 0.0)

        init_outbuf()
        load_xr(d0)

        init_carry = (
            d0,
            d0,
            tuple(jnp.float32(-1e30) for _ in range(nh)),
            tuple(zero_v for _ in range(nh)),
            tuple(zero_v for _ in range(K)),
        )

        def make_edge_body(pb, slot):
          def edge_body(j, carry):
            d_cur, w_lo, ms, dsums, accs = carry
            p = pb + j
            pvalid = (p >= p0) & (p < p1)
            dv = dst_v[slot, pl.ds(j, 16)][0]
            is_new = pvalid & (dv != d_cur)

            @pl.when(is_new)
            def _():
                finalize(d_cur, w_lo, dsums, accs)

            n_adv = jnp.where(is_new, (dv - w_lo) // oblk, 0)
            w_lo = lax.fori_loop(0, n_adv, lambda _, wl: wbody(wl), w_lo)

            d_cur = jnp.where(is_new, dv, d_cur)
            ms = tuple(jnp.where(is_new, jnp.float32(-1e30), m) for m in ms)
            row = d_cur - w_lo

            parts = [zero_v for _ in range(nh)]
            xls = []
            for kk in range(K):
                sl = pl.ds(16 * kk, 16)
                xlk = xlg_v[slot, j, sl]
                xls.append(xlk)
                z = xlk + e_v[slot, j, sl] + xr_v[row, sl]
                parts[kk // cps] = parts[kk // cps] + att_regs[kk] * jnp.abs(z)

            new_ms, svs, wvs = [], [], []
            for s in range(nh):
                lv = lin_v[slot, s, pl.ds(j, 16)][0]
                l = 0.6 * lv + 0.4 * jnp.sum(parts[s])
                l = jnp.where(pvalid, l, jnp.float32(-1e30))
                mn = jnp.maximum(ms[s], l)
                sv = jnp.exp(lax.broadcast(ms[s] - mn, (16,)))
                wv = jnp.exp(lax.broadcast(l - mn, (16,)))
                wv = jnp.where(pvalid, wv, zero_v)
                new_ms.append(mn)
                svs.append(sv)
                wvs.append(wv)

            dsums = tuple(dsums[s] * svs[s] + wvs[s] for s in range(nh))
            accs = tuple(accs[kk] * svs[kk // cps] + wvs[kk // cps] * xls[kk]
                         for kk in range(K))
            return (d_cur, w_lo, tuple(new_ms), dsums, accs)
          return edge_body

        def issue_block(bi, slot):
            pb = (blk0 + bi) * B
            pltpu.sync_copy(srcs_hbm.at[pl.ds(pb, B)], src_v.at[slot])
            pltpu.sync_copy(dsts_hbm.at[pl.ds(pb, B)],
                            dst_v.at[slot, pl.ds(0, B)])
            for s in range(nh):
                pltpu.sync_copy(lin_hbm.at[pl.ds((h * nh + s) * E + pb, B)],
                                lin_v.at[slot, s, pl.ds(0, B)])
            pltpu.async_copy(e_hbm.at[h, pl.ds(pb, B)], e_v.at[slot],
                             sem1.at[slot])
            pltpu.async_copy(xl_hbm.at[h].at[src_v.at[slot]], xlg_v.at[slot],
                             sem2.at[slot])

        def wait_block(bi, slot):
            pb = (blk0 + bi) * B
            pltpu.make_async_copy(e_hbm.at[h, pl.ds(pb, B)], e_v.at[slot],
                                  sem1.at[slot]).wait()
            pltpu.make_async_copy(xl_hbm.at[h].at[src_v.at[slot]],
                                  xlg_v.at[slot], sem2.at[slot]).wait()

        def block_body(bi, carry):
            pb = (blk0 + bi) * B
            slot = lax.rem(bi, 2)
            wait_block(bi, slot)

            @pl.when(bi + 1 < nblk)
            def _():
                issue_block(bi + 1, 1 - slot)

            return lax.fori_loop(0, B, make_edge_body(pb, slot), carry)

        @pl.when(nblk > 0)
        def _():
            issue_block(0, 0)

        d_cur, w_lo, ms, dsums, accs = lax.fori_loop(
            0, nblk, block_body, init_carry)

        finalize(d_cur, w_lo, dsums, accs)
        lax.fori_loop(0, (d1 - w_lo) // oblk, lambda _, wl: wbody(wl), w_lo)

    return k(xl, xr, e, lin, srcs, dsts, bounds, att, b)


def kernel(x, edge_index, edge_attr, batch, Wl1, Wr1, We1, att1, b1,
           Wl2, Wr2, We2, att2, b2, W_fc1, b_fc1, W_fc2, b_fc2):
    src = edge_index[0]
    dst = edge_index[1]
    ids = lax.iota(jnp.int32, E)
    dst_s, src_s, perm = lax.sort((dst, src, ids), num_keys=1)

    bounds1 = jnp.searchsorted(dst_s, jnp.arange(9) * 1280).astype(jnp.int32)
    bounds1 = jnp.pad(bounds1, (0, 32 - 9), constant_values=E)
    bounds2 = jnp.searchsorted(dst_s, jnp.arange(33) * 320).astype(jnp.int32)
    bounds2 = jnp.pad(bounds2, (0, 48 - 33), constant_values=E)

    ea_s = jnp.take(edge_attr, perm, axis=0)
    de = edge_attr.shape[1]

    wla1 = (Wl1.reshape(D, H1, C1) * att1[None]).sum(-1)
    wra1 = (Wr1.reshape(D, H1, C1) * att1[None]).sum(-1)
    wea1 = (We1.reshape(de, H1, C1) * att1[None]).sum(-1)
    lin1 = (jnp.take(x @ wla1, src_s, axis=0)
            + jnp.take(x @ wra1, dst_s, axis=0) + ea_s @ wea1).T.reshape(-1)

    xl1, xr1 = _mm_nodes1(x, Wl1, Wr1)
    e1 = _mm_edges(ea_s, We1, H1, C1)
    h1 = _gat_sc(xl1, xr1, e1, lin1, src_s, dst_s, bounds1, att1,
                 b1.reshape(H1, C1),
                 n_ranges=8, rng_sz=1280, oblk=64, n_out_rows=N_PAD1)

    xl2, xr2 = _mm_nodes2(h1, Wl2, Wr2)
    e2 = _mm_edges(ea_s, We2, 1, F2)
    wla2 = (Wl2.reshape(F1, H2, C2) * att2[None]).sum(-1).reshape(H1, C1, H2)
    wra2 = (Wr2.reshape(F1, H2, C2) * att2[None]).sum(-1).reshape(H1, C1, H2)
    wea2 = (We2.reshape(de, H2, C2) * att2[None]).sum(-1)
    alx2 = jnp.einsum('hnc,hcj->nj', h1[:, :N], wla2)
    axr2 = jnp.einsum('hnc,hcj->nj', h1[:, :N], wra2)
    lin2 = (jnp.take(alx2, src_s, axis=0)
            + jnp.take(axr2, dst_s, axis=0) + ea_s @ wea2).T.reshape(-1)
    h2 = _gat_sc(xl2.reshape(1, N, F2), xr2.reshape(1, N, F2), e2, lin2,
                 src_s, dst_s, bounds2, att2.reshape(1, F2),
                 b2.reshape(1, F2),
                 n_ranges=32, rng_sz=320, oblk=64, n_out_rows=N_PAD2)

    batch_pad = jnp.pad(batch, (0, N_PAD2 - N),
                        constant_values=G).reshape(N_PAD2, 1)
    return _pool_fc(h2[0], batch_pad, W_fc1, b_fc1, W_fc2, b_fc2)


# speedup vs baseline: 1.4422x; 1.4422x over previous
import functools

import jax
import jax.numpy as jnp
from jax import lax
from jax.experimental import pallas as pl
from jax.experimental.pallas import tpu as pltpu
from jax.experimental.pallas import tpu_sc as plsc

N = 10000
E = 160000
D = 128
G = 32
H1, C1 = 4, 256
H2, C2 = 4, 64
F1 = H1 * C1
F2 = H2 * C2
B = 64
N_PAD1 = 10240
N_PAD2 = 10240


def _mm_nodes1_body(x_ref, wl_ref, wr_ref, ol_ref, or_ref):
    x = x_ref[...]
    ol_ref[0] = jnp.dot(x, wl_ref[...], preferred_element_type=jnp.float32)
    or_ref[0] = jnp.dot(x, wr_ref[...], preferred_element_type=jnp.float32)


def _mm_nodes1(x, Wl, Wr):
    bn = 2000
    return pl.pallas_call(
        _mm_nodes1_body,
        grid=(H1, N // bn),
        in_specs=[
            pl.BlockSpec((bn, D), lambda h, i: (i, 0)),
            pl.BlockSpec((D, C1), lambda h, i: (0, h)),
            pl.BlockSpec((D, C1), lambda h, i: (0, h)),
        ],
        out_specs=[
            pl.BlockSpec((1, bn, C1), lambda h, i: (h, i, 0)),
            pl.BlockSpec((1, bn, C1), lambda h, i: (h, i, 0)),
        ],
        out_shape=[
            jax.ShapeDtypeStruct((H1, N, C1), jnp.float32),
            jax.ShapeDtypeStruct((H1, N, C1), jnp.float32),
        ],
    )(x, Wl, Wr)


def _mm_nodes2_body(h_ref, wl_ref, wr_ref, ol_ref, or_ref):
    accl = jnp.zeros((h_ref.shape[1], F2), jnp.float32)
    accr = jnp.zeros((h_ref.shape[1], F2), jnp.float32)
    for hh in range(H1):
        blk = h_ref[hh]
        accl += jnp.dot(blk, wl_ref[hh], preferred_element_type=jnp.float32)
        accr += jnp.dot(blk, wr_ref[hh], preferred_element_type=jnp.float32)
    ol_ref[...] = accl
    or_ref[...] = accr


def _mm_nodes2(h1, Wl, Wr):
    bn = 1000
    wl = Wl.reshape(H1, C1, F2)
    wr = Wr.reshape(H1, C1, F2)
    return pl.pallas_call(
        _mm_nodes2_body,
        grid=(N // bn,),
        in_specs=[
            pl.BlockSpec((H1, bn, C1), lambda i: (0, i, 0)),
            pl.BlockSpec((H1, C1, F2), lambda i: (0, 0, 0)),
            pl.BlockSpec((H1, C1, F2), lambda i: (0, 0, 0)),
        ],
        out_specs=[
            pl.BlockSpec((bn, F2), lambda i: (i, 0)),
            pl.BlockSpec((bn, F2), lambda i: (i, 0)),
        ],
        out_shape=[
            jax.ShapeDtypeStruct((N, F2), jnp.float32),
            jax.ShapeDtypeStruct((N, F2), jnp.float32),
        ],
    )(h1, wl, wr)


def _mm_edges_body(ea_ref, w_ref, o_ref):
    o_ref[0] = jnp.dot(ea_ref[...], w_ref[...],
                       preferred_element_type=jnp.float32)


def _mm_edges(ea, We, ht, c):
    be = 2000
    de = ea.shape[1]
    return pl.pallas_call(
        _mm_edges_body,
        grid=(ht, E // be),
        in_specs=[
            pl.BlockSpec((be, de), lambda h, i: (i, 0)),
            pl.BlockSpec((de, c), lambda h, i: (0, h)),
        ],
        out_specs=pl.BlockSpec((1, be, c), lambda h, i: (h, i, 0)),
        out_shape=jax.ShapeDtypeStruct((ht, E, c), jnp.float32),
    )(ea, We)


def _pool_fc_body(h_ref, bat_ref, w1_ref, b1_ref, w2_ref, b2_ref, o_ref):
    hv = h_ref[...]
    bat = bat_ref[...]
    gids = lax.broadcasted_iota(jnp.int32, (N_PAD2, G), 1)
    S = (bat == gids).astype(jnp.float32)
    xadd = lax.dot_general(S, hv, (((0,), (0,)), ((), ())),
                           preferred_element_type=jnp.float32)
    cnt = jnp.sum(S, axis=0)[:, None]
    xmean = xadd / jnp.maximum(cnt, 1.0)
    p = jnp.concatenate([xadd, xmean], axis=1)
    z = jnp.maximum(
        jnp.dot(p, w1_ref[...], preferred_element_type=jnp.float32)
        + b1_ref[...], 0.0)
    o_ref[...] = (jnp.dot(z, w2_ref[...], preferred_element_type=jnp.float32)
                  + b2_ref[...])


def _pool_fc(h2_pad, batch_pad, W_fc1, b_fc1, W_fc2, b_fc2):
    return pl.pallas_call(
        _pool_fc_body,
        in_specs=[
            pl.BlockSpec((N_PAD2, F2), lambda: (0, 0)),
            pl.BlockSpec((N_PAD2, 1), lambda: (0, 0)),
            pl.BlockSpec(W_fc1.shape, lambda: (0, 0)),
            pl.BlockSpec((1, 64), lambda: (0, 0)),
            pl.BlockSpec(W_fc2.shape, lambda: (0, 0)),
            pl.BlockSpec((1, 1), lambda: (0, 0)),
        ],
        out_specs=pl.BlockSpec((G, 1), lambda: (0, 0)),
        out_shape=jax.ShapeDtypeStruct((G, 1), jnp.float32),
    )(h2_pad, batch_pad, W_fc1, b_fc1.reshape(1, 64), W_fc2,
      b_fc2.reshape(1, 1))


def _gat_sc(xl, xr, e, srcs, dsts, bounds, att, b, *,
            n_ranges, rng_sz, oblk, n_out_rows):
    HT, NN, CR = xl.shape
    att = att.reshape(HT, 1, CR)
    b = b.reshape(HT, 1, CR)
    K = CR // 16
    nh = 4 // HT
    cps = K // nh
    n_bpad = bounds.shape[0]
    mesh = plsc.VectorSubcoreMesh(core_axis_name="c", subcore_axis_name="s")

    @functools.partial(
        pl.kernel,
        out_type=jax.ShapeDtypeStruct((HT, n_out_rows, CR), jnp.float32),
        mesh=mesh,
        compiler_params=pltpu.CompilerParams(needs_layout_passes=False),
        scratch_types=[
            pltpu.VMEM((2, B), jnp.int32),
            pltpu.VMEM((2, B + 16), jnp.int32),
            pltpu.VMEM((2, B, CR), jnp.float32),
            pltpu.VMEM((2, B, CR), jnp.float32),
            pltpu.VMEM((oblk, CR), jnp.float32),
            pltpu.VMEM((oblk, CR), jnp.float32),
            pltpu.VMEM((1, CR), jnp.float32),
            pltpu.VMEM((1, CR), jnp.float32),
            pltpu.VMEM((n_bpad,), jnp.int32),
            pltpu.SemaphoreType.DMA((2,)),
            pltpu.SemaphoreType.DMA((2,)),
        ],
    )
    def k(xl_hbm, xr_hbm, e_hbm, srcs_hbm, dsts_hbm, bounds_hbm,
          att_hbm, b_hbm, out_hbm, src_v, dst_v, e_v, xlg_v, xr_v,
          out_v, att_v, b_v, bounds_v, sem1, sem2):
        wid = lax.axis_index("s") * 2 + lax.axis_index("c")
        h = wid // n_ranges
        r = wid % n_ranges
        d0 = r * rng_sz
        d1 = d0 + rng_sz

        pltpu.sync_copy(bounds_hbm, bounds_v)
        pltpu.sync_copy(att_hbm.at[h], att_v)
        pltpu.sync_copy(b_hbm.at[h], b_v)
        p0 = bounds_v[pl.ds(r, 16)][0]
        p1 = bounds_v[pl.ds(r + 1, 16)][0]
        blk0 = p0 // B
        nblk = (p1 + B - 1) // B - blk0

        att_regs = [att_v[0, pl.ds(16 * kk, 16)] for kk in range(K)]
        zero_v = jnp.zeros((16,), jnp.float32)

        def init_outbuf():
            def ib(i, c):
                for kk in range(K):
                    sl = pl.ds(16 * kk, 16)
                    out_v[i, sl] = jnp.maximum(b_v[0, sl], 0.0)
                return c
            lax.fori_loop(0, oblk, ib, 0)

        def load_xr(wl):
            start = pl.multiple_of(jnp.minimum(wl, NN - oblk), 8)
            pltpu.sync_copy(xr_hbm.at[h, pl.ds(start, oblk)], xr_v)

        def flush(wl):
            wl = pl.multiple_of(wl, 8)
            pltpu.sync_copy(out_v, out_hbm.at[h, pl.ds(wl, oblk)])

        def wbody(wl):
            flush(wl)
            init_outbuf()
            load_xr(wl + oblk)
            return wl + oblk

        def finalize(d_cur, w_lo, dsums, accs):
            row = d_cur - w_lo
            invs = [1.0 / (dsums[s] + 1e-16) for s in range(nh)]
            for kk in range(K):
                sl = pl.ds(16 * kk, 16)
                val = accs[kk] * invs[kk // cps] + b_v[0, sl]
                out_v[row, sl] = jnp.maximum(val, 0.0)

        init_outbuf()
        load_xr(d0)

        init_carry = (
            d0,
            d0,
            tuple(jnp.float32(-1e30) for _ in range(nh)),
            tuple(zero_v for _ in range(nh)),
            tuple(zero_v for _ in range(K)),
        )

        def make_edge_body(pb, slot):
          def edge_body(j, carry):
            d_cur, w_lo, ms, dsums, accs = carry
            p = pb + j
            pvalid = (p >= p0) & (p < p1)
            dv = dst_v[slot, pl.ds(j, 16)][0]
            is_new = pvalid & (dv != d_cur)

            @pl.when(is_new)
            def _():
                finalize(d_cur, w_lo, dsums, accs)

            n_adv = jnp.where(is_new, (dv - w_lo) // oblk, 0)
            w_lo = lax.fori_loop(0, n_adv, lambda _, wl: wbody(wl), w_lo)

            d_cur = jnp.where(is_new, dv, d_cur)
            ms = tuple(jnp.where(is_new, jnp.float32(-1e30), m) for m in ms)
            row = d_cur - w_lo

            parts = [zero_v for _ in range(nh)]
            xls = []
            for kk in range(K):
                sl = pl.ds(16 * kk, 16)
                xlk = xlg_v[slot, j, sl]
                xls.append(xlk)
                z = xlk + e_v[slot, j, sl] + xr_v[row, sl]
                z = jnp.maximum(z, 0.2 * z)
                parts[kk // cps] = parts[kk // cps] + att_regs[kk] * z

            new_ms, svs, wvs = [], [], []
            for s in range(nh):
                l = jnp.sum(parts[s])
                l = jnp.where(pvalid, l, jnp.float32(-1e30))
                mn = jnp.maximum(ms[s], l)
                sv = jnp.exp(lax.broadcast(ms[s] - mn, (16,)))
                wv = jnp.exp(lax.broadcast(l - mn, (16,)))
                wv = jnp.where(pvalid, wv, zero_v)
                new_ms.append(mn)
                svs.append(sv)
                wvs.append(wv)

            dsums = tuple(dsums[s] * svs[s] + wvs[s] for s in range(nh))
            accs = tuple(accs[kk] * svs[kk // cps] + wvs[kk // cps] * xls[kk]
                         for kk in range(K))
            return (d_cur, w_lo, tuple(new_ms), dsums, accs)
          return edge_body

        def issue_block(bi, slot):
            pb = (blk0 + bi) * B
            pltpu.sync_copy(srcs_hbm.at[pl.ds(pb, B)], src_v.at[slot])
            pltpu.sync_copy(dsts_hbm.at[pl.ds(pb, B)],
                            dst_v.at[slot, pl.ds(0, B)])
            pltpu.async_copy(e_hbm.at[h, pl.ds(pb, B)], e_v.at[slot],
                             sem1.at[slot])
            pltpu.async_copy(xl_hbm.at[h].at[src_v.at[slot]], xlg_v.at[slot],
                             sem2.at[slot])

        def wait_block(bi, slot):
            pb = (blk0 + bi) * B
            pltpu.make_async_copy(e_hbm.at[h, pl.ds(pb, B)], e_v.at[slot],
                                  sem1.at[slot]).wait()
            pltpu.make_async_copy(xl_hbm.at[h].at[src_v.at[slot]],
                                  xlg_v.at[slot], sem2.at[slot]).wait()

        def block_body(bi, carry):
            pb = (blk0 + bi) * B
            slot = lax.rem(bi, 2)
            wait_block(bi, slot)

            @pl.when(bi + 1 < nblk)
            def _():
                issue_block(bi + 1, 1 - slot)

            return lax.fori_loop(0, B, make_edge_body(pb, slot), carry)

        @pl.when(nblk > 0)
        def _():
            issue_block(0, 0)

        d_cur, w_lo, ms, dsums, accs = lax.fori_loop(
            0, nblk, block_body, init_carry)

        finalize(d_cur, w_lo, dsums, accs)
        lax.fori_loop(0, (d1 - w_lo) // oblk, lambda _, wl: wbody(wl), w_lo)

    return k(xl, xr, e, srcs, dsts, bounds, att, b)


def kernel(x, edge_index, edge_attr, batch, Wl1, Wr1, We1, att1, b1,
           Wl2, Wr2, We2, att2, b2, W_fc1, b_fc1, W_fc2, b_fc2):
    src = edge_index[0]
    dst = edge_index[1]
    ids = lax.iota(jnp.int32, E)
    dst_s, src_s, perm = lax.sort((dst, src, ids), num_keys=1)

    bounds1 = jnp.searchsorted(dst_s, jnp.arange(9) * 1280).astype(jnp.int32)
    bounds1 = jnp.pad(bounds1, (0, 32 - 9), constant_values=E)
    bounds2 = jnp.searchsorted(dst_s, jnp.arange(33) * 320).astype(jnp.int32)
    bounds2 = jnp.pad(bounds2, (0, 48 - 33), constant_values=E)

    ea_s = jnp.take(edge_attr, perm, axis=0)

    xl1, xr1 = _mm_nodes1(x, Wl1, Wr1)
    e1 = _mm_edges(ea_s, We1, H1, C1)
    h1 = _gat_sc(xl1, xr1, e1, src_s, dst_s, bounds1, att1,
                 b1.reshape(H1, C1),
                 n_ranges=8, rng_sz=1280, oblk=64, n_out_rows=N_PAD1)

    xl2, xr2 = _mm_nodes2(h1, Wl2, Wr2)
    e2 = _mm_edges(ea_s, We2, 1, F2)
    h2 = _gat_sc(xl2.reshape(1, N, F2), xr2.reshape(1, N, F2), e2,
                 src_s, dst_s, bounds2, att2.reshape(1, F2),
                 b2.reshape(1, F2),
                 n_ranges=32, rng_sz=320, oblk=64, n_out_rows=N_PAD2)

    batch_pad = jnp.pad(batch, (0, N_PAD2 - N),
                        constant_values=G).reshape(N_PAD2, 1)
    return _pool_fc(h2[0], batch_pad, W_fc1, b_fc1, W_fc2, b_fc2)
